# baseline (speedup 1.0000x reference)
import functools

import jax
import jax.numpy as jnp
from jax import lax
from jax.experimental import pallas as pl
from jax.experimental.pallas import tpu as pltpu
from jax.experimental.pallas import tpu_sc as plsc

BATCH = 4096
SEQ = 200
EMBED = 64
VOCAB = 1000000
ROW = 128
VG = -(-VOCAB // ROW)
VPAD = VG * ROW

NC = 2
NS = 16
NW = NC * NS

A_SLABS = -(-VG // NW)
LANES_PER_W = BATCH // NW


def _iotas():
    return [lax.iota(jnp.int32, 16) + 16 * k for k in range(8)]


@functools.cache
def _build_stage_table():
    mesh = plsc.VectorSubcoreMesh(core_axis_name="c", subcore_axis_name="s")

    @functools.partial(
        pl.kernel,
        mesh=mesh,
        out_type=jax.ShapeDtypeStruct((VPAD, ROW), jnp.float32),
        scratch_types=[
            pltpu.VMEM((EMBED, ROW), jnp.float32),
            pltpu.VMEM((EMBED, ROW), jnp.float32),
            pltpu.VMEM((ROW, ROW), jnp.float32),
            pltpu.VMEM((ROW, ROW), jnp.float32),
            pltpu.SemaphoreType.DMA,
            pltpu.SemaphoreType.DMA,
            pltpu.SemaphoreType.DMA,
            pltpu.SemaphoreType.DMA,
        ],
        compiler_params=pltpu.CompilerParams(
            disable_bounds_checks=True, needs_layout_passes=False),
    )
    def _stage(tab_t, out, in0, in1, tr0, tr1, si0, si1, so0, so1):
        wid = lax.axis_index("s") * NC + lax.axis_index("c")
        inb = (in0, in1)
        trb = (tr0, tr1)
        si = (si0, si1)
        so = (so0, so1)
        iotas = _iotas()

        def in_range(t):
            return (t < A_SLABS) & (wid * A_SLABS + t < VG)

        def start_in(t, b):
            vg = wid * A_SLABS + t
            pltpu.async_copy(
                tab_t.at[pl.ds(0, EMBED), pl.ds(vg * ROW, ROW)], inb[b], si[b])

        def wait_in(b):
            pltpu.make_async_copy(
                tab_t.at[pl.ds(0, EMBED), pl.ds(0, ROW)], inb[b], si[b]).wait()

        def start_out(t, b):
            vg = wid * A_SLABS + t
            pltpu.async_copy(trb[b], out.at[pl.ds(vg * ROW, ROW)], so[b])

        def wait_out(b):
            pltpu.make_async_copy(trb[b], out.at[pl.ds(0, ROW)], so[b]).wait()

        def transpose(b):
            @plsc.parallel_loop(0, EMBED, unroll=16)
            def _(e):
                esplat = lax.broadcast(e, (16,))
                for k in range(8):
                    vals = inb[b][e, pl.ds(16 * k, 16)]
                    plsc.store_scatter(trb[b], [iotas[k], esplat], vals)

        start_in(0, 0)

        def body(t2, carry):
            for b in range(2):
                t = 2 * t2 + b
                ok = in_range(t)

                @pl.when(ok)
                def _():
                    wait_in(b)

                @pl.when(in_range(t + 1))
                def _():
                    start_in(t + 1, 1 - b)

                @pl.when(ok & (t >= 2))
                def _():
                    wait_out(b)

                @pl.when(ok)
                def _():
                    transpose(b)
                    start_out(t, b)
            return carry

        lax.fori_loop(0, -(-A_SLABS // 2), body, 0)
        wait_out(0)
        wait_out(1)

    return _stage


@functools.cache
def _build_gather():
    mesh = plsc.VectorSubcoreMesh(core_axis_name="c", subcore_axis_name="s")

    @functools.partial(
        pl.kernel,
        mesh=mesh,
        out_type=jax.ShapeDtypeStruct((SEQ, EMBED, BATCH), jnp.float32),
        scratch_types=[
            pltpu.VMEM((SEQ, LANES_PER_W), jnp.int32),
            pltpu.VMEM((LANES_PER_W, ROW), jnp.float32),
            pltpu.VMEM((LANES_PER_W, ROW), jnp.float32),
            pltpu.VMEM((EMBED, LANES_PER_W), jnp.float32),
            pltpu.VMEM((EMBED, LANES_PER_W), jnp.float32),
            pltpu.SemaphoreType.DMA,
            pltpu.SemaphoreType.DMA,
            pltpu.SemaphoreType.DMA,
            pltpu.SemaphoreType.DMA,
        ],
        compiler_params=pltpu.CompilerParams(
            disable_bounds_checks=True, needs_layout_passes=False),
    )
    def _gather(idx2d, table, out, idx_v, g0, g1, tr0, tr1, sg0, sg1, so0, so1):
        wid = lax.axis_index("s") * NC + lax.axis_index("c")
        lane0 = wid * LANES_PER_W
        gb = (g0, g1)
        trb = (tr0, tr1)
        sg = (sg0, sg1)
        so = (so0, so1)
        iotas = _iotas()

        pltpu.sync_copy(
            idx2d.at[pl.ds(0, SEQ), pl.ds(lane0, LANES_PER_W)], idx_v)

        def start_gather(s, b):
            pltpu.async_copy(table.at[idx_v.at[s]], gb[b], sg[b])

        def wait_gather(b):
            pltpu.make_async_copy(
                table.at[pl.ds(0, LANES_PER_W)], gb[b], sg[b]).wait()

        def start_out(s, b):
            pltpu.async_copy(
                trb[b],
                out.at[s, pl.ds(0, EMBED), pl.ds(lane0, LANES_PER_W)],
                so[b])

        def wait_out(b):
            pltpu.make_async_copy(
                trb[b],
                out.at[0, pl.ds(0, EMBED), pl.ds(lane0, LANES_PER_W)],
                so[b]).wait()

        def transpose(b):
            @plsc.parallel_loop(0, EMBED, unroll=16)
            def _(e):
                esplat = lax.broadcast(e, (16,))
                for k in range(8):
                    vals = plsc.load_gather(gb[b], [iotas[k], esplat])
                    trb[b][e, pl.ds(16 * k, 16)] = vals

        start_gather(0, 0)

        def body(t, carry):
            for b in range(2):
                s = 2 * t + b
                wait_gather(b)

                @pl.when(s + 1 < SEQ)
                def _():
                    start_gather(s + 1, 1 - b)

                @pl.when(s >= 2)
                def _():
                    wait_out(b)

                transpose(b)
                start_out(s, b)
            return carry

        lax.fori_loop(0, SEQ // 2, body, 0)
        wait_out(0)
        wait_out(1)

    return _gather


def kernel(token_matrix, table):
    idx2d = token_matrix.T.astype(jnp.int32)
    table128 = _build_stage_table()(table.T)
    out3 = _build_gather()(idx2d, table128)
    return jnp.transpose(out3, (2, 0, 1))


# speedup vs baseline: 1.2252x; 1.2252x over previous
import functools

import jax
import jax.numpy as jnp
from jax import lax
from jax.experimental import pallas as pl
from jax.experimental.pallas import tpu as pltpu
from jax.experimental.pallas import tpu_sc as plsc

BATCH = 4096
SEQ = 200
EMBED = 64
NTOK = BATCH * SEQ

NC = 2
NS = 16
NW = NC * NS

IDX_W = 128
CHUNK = 512
DMAS = CHUNK // IDX_W
TOK_PER_W = NTOK // NW
STEPS = TOK_PER_W // CHUNK
IDX_ROWS_PER_W = TOK_PER_W // IDX_W

@functools.cache
def _build_sc_gather():
    mesh = plsc.VectorSubcoreMesh(core_axis_name="c", subcore_axis_name="s")

    @functools.partial(
        pl.kernel,
        mesh=mesh,
        out_type=jax.ShapeDtypeStruct((NTOK, EMBED), jnp.float32),
        scratch_types=[
            pltpu.VMEM((IDX_ROWS_PER_W, IDX_W), jnp.int32),
            pltpu.VMEM((2 * CHUNK, EMBED), jnp.float32),
            pltpu.SemaphoreType.DMA,
            pltpu.SemaphoreType.DMA,
            pltpu.SemaphoreType.DMA,
            pltpu.SemaphoreType.DMA,
        ],
        compiler_params=pltpu.CompilerParams(use_tc_tiling_on_sc=False),
    )
    def _sc_gather(idx_hbm, table_hbm, out_hbm, idx_v, rows_v, sg0, sg1, so0, so1):
        wid = lax.axis_index("s") * NC + lax.axis_index("c")
        row_base = wid * IDX_ROWS_PER_W
        out_base = wid * TOK_PER_W
        sg = (sg0, sg1)
        so = (so0, so1)

        pltpu.sync_copy(idx_hbm.at[pl.ds(row_base, IDX_ROWS_PER_W)], idx_v)

        def gathers(s, b):
            cps = [
                pltpu.async_copy(
                    table_hbm.at[idx_v.at[s * DMAS + j]],
                    rows_v.at[pl.ds(b * CHUNK + j * IDX_W, IDX_W)],
                    sg[b],
                )
                for j in range(DMAS)
            ]
            for cp in cps:
                cp.wait()

        def start_out(s, b):
            pltpu.async_copy(
                rows_v.at[pl.ds(b * CHUNK, CHUNK)],
                out_hbm.at[pl.ds(out_base + s * CHUNK, CHUNK)],
                so[b],
            )

        def wait_out(b):
            pltpu.make_async_copy(
                rows_v.at[pl.ds(b * CHUNK, CHUNK)],
                out_hbm.at[pl.ds(out_base, CHUNK)],
                so[b],
            ).wait()

        for b in range(2):
            gathers(b, b)
            start_out(b, b)

        def body(k, carry):
            for b in range(2):
                s = 2 * k + b
                wait_out(b)
                gathers(s, b)
                start_out(s, b)
            return carry

        lax.fori_loop(1, STEPS // 2, body, 0)
        wait_out(0)
        wait_out(1)

    return _sc_gather


def kernel(token_matrix, table):
    idx = token_matrix.astype(jnp.int32).reshape(NTOK // IDX_W, IDX_W)
    out = _build_sc_gather()(idx, table)
    return out.reshape(BATCH, SEQ, EMBED)
